# Initial kernel scaffold; baseline (speedup 1.0000x reference)
import jax
import jax.numpy as jnp
from jax.experimental import pallas as pl


def kernel(x, edge_index, batch, ln_g, ln_b, bn1_g, bn1_b, bn1_m, bn1_v, cnn_W, cnn_b, bn2_g, bn2_b, bn2_m, bn2_v, pre_W, pre_b, gat_W, att_src, att_dst, gat_bias, bn3_g, bn3_b, bn3_m, bn3_v, mlp_W1, mlp_b1, mlp_W2, mlp_b2):
    raise NotImplementedError("write your pallas kernel here")


# speedup vs baseline: 26.3857x; 26.3857x over previous
import functools

import jax
import jax.numpy as jnp
from jax import lax
from jax.experimental import pallas as pl
from jax.experimental.pallas import tpu as pltpu
from jax.experimental.pallas import tpu_sc as plsc

_N = 50000
_H = 4
_C = 32
_G = 256
_EPS = 1e-5
_R = 400
_GRID = _N // _R

_NSC = 2
_NTILE = 16
_CHUNK = 1024
_CPT = 52
_NCH = _NTILE * _CPT
_EPAD = _NCH * _CHUNK
_RPT = 3128
_NSP = _NTILE * _RPT


def _pre_body(x_ref, wa_ref, wb_ref, b_ref, gw_ref, as_w_ref, ad_w_ref,
              z_ref, as_ref, ad_ref, m_ref):
    i = pl.program_id(0)
    xb = x_ref[...]
    xc = xb[:, :512]
    xf = xb[:, 512:]
    mu = jnp.mean(xf, axis=-1, keepdims=True)
    var = jnp.mean((xf - mu) ** 2, axis=-1, keepdims=True)
    xfn = (xf - mu) / jnp.sqrt(var + _EPS)
    h = jnp.dot(xc, wa_ref[...], preferred_element_type=jnp.float32)
    h = h + jnp.dot(xfn, wb_ref[...], preferred_element_type=jnp.float32)
    h = jnp.maximum(h + b_ref[...], 0.0)
    z = jnp.dot(h, gw_ref[...], preferred_element_type=jnp.float32)
    for hd in range(_H):
        z_ref[hd] = z[:, hd * _C:(hd + 1) * _C]
    a_s = jnp.dot(h, as_w_ref[...], preferred_element_type=jnp.float32)
    a_d = jnp.dot(h, ad_w_ref[...], preferred_element_type=jnp.float32)
    as_ref[...] = a_s
    ad_ref[...] = a_d
    mx = jnp.concatenate([jnp.max(a_s, axis=0)[:_H], jnp.max(a_d, axis=0)[:_H]])
    mx = mx.reshape(1, 2 * _H)

    @pl.when(i == 0)
    def _():
        m_ref[...] = mx

    @pl.when(i != 0)
    def _():
        m_ref[...] = jnp.maximum(m_ref[...], mx)


def _pre_call(x, w_a, w_b, b_ab, gat_W, as_w, ad_w):
    full = lambda shape: pl.BlockSpec(shape, lambda i: (0,) * len(shape))
    return pl.pallas_call(
        _pre_body,
        grid=(_GRID,),
        in_specs=[
            pl.BlockSpec((_R, 576), lambda i: (i, 0)),
            full((512, 32)), full((64, 32)), full((1, 32)),
            full((32, _H * _C)), full((32, 16)), full((32, 16)),
        ],
        out_specs=[
            pl.BlockSpec((_H, _R, _C), lambda i: (0, i, 0)),
            pl.BlockSpec((_R, 16), lambda i: (i, 0)),
            pl.BlockSpec((_R, 16), lambda i: (i, 0)),
            pl.BlockSpec((1, 2 * _H), lambda i: (0, 0)),
        ],
        out_shape=[
            jax.ShapeDtypeStruct((_H, _N, _C), jnp.float32),
            jax.ShapeDtypeStruct((_N, 16), jnp.float32),
            jax.ShapeDtypeStruct((_N, 16), jnp.float32),
            jax.ShapeDtypeStruct((1, 2 * _H), jnp.float32),
        ],
    )(x, w_a, w_b, b_ab, gat_W, as_w, ad_w)


def _edge_body(ereal, src_h, dst_h, as_h, ad_h, zf_h, mrep_h,
               accum_o, s_o, exsp_o,
               src_v, dst_v, srcz_v, as_v, ad_v, exh_v, exh1_v,
               zs_v, zrows_v, zbuf_v, mrep_v, sem,
               accum_sp, s_sp):
    cid = lax.axis_index("c")
    sid = lax.axis_index("s")
    iota = lax.iota(jnp.int32, 16)
    zeros16 = jnp.zeros((16,), jnp.float32)

    def _z0(e, _):
        zbuf_v[e, pl.ds(0, 16)] = zeros16
        return 0
    lax.fori_loop(0, _CHUNK, _z0, 0)

    def _z1(j, _):
        zs_v[pl.ds(j * 16, 16)] = zeros16
        return 0
    lax.fori_loop(0, _CHUNK // 16, _z1, 0)

    pltpu.sync_copy(mrep_h, mrep_v)

    r0 = sid * _RPT

    def zero_accum():
        for q in range(3):
            pltpu.sync_copy(zbuf_v, accum_sp.at[pl.ds(r0 + q * 1024, 1024), :])
        pltpu.sync_copy(zbuf_v.at[pl.ds(0, 56), :],
                        accum_sp.at[pl.ds(r0 + 3072, 56), :])

    def zero_s():
        for q in range(3):
            pltpu.sync_copy(zs_v, s_sp.at[pl.ds(r0 + q * 1024, 1024)])
        pltpu.sync_copy(zs_v.at[pl.ds(0, 56)], s_sp.at[pl.ds(r0 + 3072, 56)])

    zero_accum()
    zero_s()
    plsc.subcore_barrier()

    mrv = mrep_v[...]

    for p in range(4):
        hp, ch = p >> 1, p & 1
        head = 2 * cid + hp
        zmul_off = 2 * head * _N + ch

        def chunk_body(k, _, p=p, head=head, zmul_off=zmul_off):
            cg = sid * _CPT + k
            pltpu.sync_copy(src_h.at[cg], src_v)
            pltpu.sync_copy(dst_h.at[cg], dst_v)

            if p == 0:
                descs = []
                for j in range(8):
                    descs.append(pltpu.async_copy(
                        as_h.at[src_v.at[j]],
                        as_v.at[pl.ds(j * 128, 128), :], sem))
                    descs.append(pltpu.async_copy(
                        ad_h.at[dst_v.at[j]],
                        ad_v.at[pl.ds(j * 128, 128), :], sem))
                for d in descs:
                    d.wait()

                hm0 = iota == head
                hm1 = iota == (head + 1)

                def _cex(e, _):
                    a16 = as_v[e, pl.ds(0, 16)]
                    d16 = ad_v[e, pl.ds(0, 16)]
                    t = a16 + d16
                    t = jnp.where(t < 0.0, t * jnp.float32(0.2), t)
                    ex = jnp.exp(t - mrv)
                    eidx = jnp.broadcast_to(e, (16,))
                    ok = (cg * _CHUNK + eidx) < ereal
                    ex = jnp.where(ok, ex, 0.0)
                    plsc.store_scatter(exh_v, [eidx], ex, mask=hm0)
                    plsc.store_scatter(exh1_v, [eidx], ex, mask=hm1)
                    return 0
                lax.fori_loop(0, _CHUNK, _cex, 0)
                pltpu.sync_copy(exh_v,
                                exsp_o.at[pl.ds((head * _NCH + cg) * _CHUNK,
                                                _CHUNK)])
                pltpu.sync_copy(exh1_v,
                                exsp_o.at[pl.ds(((head + 1) * _NCH + cg)
                                                * _CHUNK, _CHUNK)])
            else:
                pltpu.sync_copy(exsp_o.at[pl.ds((head * _NCH + cg) * _CHUNK,
                                                _CHUNK)], exh_v)

            def _off(j3, _):
                sv = src_v[j3 >> 3, pl.ds((j3 & 7) * 16, 16)]
                srcz_v[j3 >> 3, pl.ds((j3 & 7) * 16, 16)] = sv * 2 + zmul_off
                return 0
            lax.fori_loop(0, _CHUNK // 16, _off, 0)

            descs = [pltpu.async_copy(zf_h.at[srcz_v.at[j]],
                                      zrows_v.at[pl.ds(j * 128, 128), :], sem)
                     for j in range(8)]
            for d in descs:
                d.wait()

            def _scale(e, _):
                exb = plsc.load_gather(exh_v, [jnp.broadcast_to(e, (16,))])
                zrows_v[e, pl.ds(0, 16)] = zrows_v[e, pl.ds(0, 16)] * exb
                return 0
            lax.fori_loop(0, _CHUNK, _scale, 0)

            descs = []
            for j in range(8):
                descs.append(pltpu.async_copy(
                    zrows_v.at[pl.ds(j * 128, 128), :],
                    accum_sp.at[dst_v.at[j]], sem, add=True))
                if p == 0 or p == 2:
                    descs.append(pltpu.async_copy(
                        exh_v.at[pl.ds(j * 128, 128)],
                        s_sp.at[dst_v.at[j]], sem, add=True))
            for d in descs:
                d.wait()
            return 0

        lax.fori_loop(0, _CPT, chunk_body, 0)
        plsc.subcore_barrier()

        for q in range(3):
            pltpu.sync_copy(accum_sp.at[pl.ds(r0 + q * 1024, 1024), :],
                            accum_o.at[head, ch, pl.ds(r0 + q * 1024, 1024), :])
        pltpu.sync_copy(accum_sp.at[pl.ds(r0 + 3072, 56), :],
                        accum_o.at[head, ch, pl.ds(r0 + 3072, 56), :])
        if p == 0 or p == 2:
            sbase = head * _NSP + r0
            for q in range(3):
                pltpu.sync_copy(s_sp.at[pl.ds(r0 + q * 1024, 1024)],
                                s_o.at[pl.ds(sbase + q * 1024, 1024)])
            pltpu.sync_copy(s_sp.at[pl.ds(r0 + 3072, 56)],
                            s_o.at[pl.ds(sbase + 3072, 56)])
        if p != 3:
            zero_accum()
            if p == 1:
                zero_s()
            plsc.subcore_barrier()


def _edge_call(ereal, src3, dst3, as4, ad4, zflat, mrep):
    mesh = plsc.VectorSubcoreMesh(core_axis_name="c", subcore_axis_name="s",
                                  num_cores=_NSC, num_subcores=_NTILE)
    kfn = pl.kernel(
        functools.partial(_edge_body, ereal),
        compiler_params=pltpu.CompilerParams(needs_layout_passes=False,
                                             use_tc_tiling_on_sc=False),
        out_type=(
            jax.ShapeDtypeStruct((_H, 2, _NSP, 16), jnp.float32),
            jax.ShapeDtypeStruct((_H * _NSP,), jnp.float32),
            jax.ShapeDtypeStruct((_H * _NCH * _CHUNK,), jnp.float32),
        ),
        mesh=mesh,
        scratch_types=[
            pltpu.VMEM((8, 128), jnp.int32),
            pltpu.VMEM((8, 128), jnp.int32),
            pltpu.VMEM((8, 128), jnp.int32),
            pltpu.VMEM((_CHUNK, 16), jnp.float32),
            pltpu.VMEM((_CHUNK, 16), jnp.float32),
            pltpu.VMEM((_CHUNK,), jnp.float32),
            pltpu.VMEM((_CHUNK,), jnp.float32),
            pltpu.VMEM((_CHUNK,), jnp.float32),
            pltpu.VMEM((_CHUNK, 16), jnp.float32),
            pltpu.VMEM((_CHUNK, 16), jnp.float32),
            pltpu.VMEM((16,), jnp.float32),
            pltpu.SemaphoreType.DMA,
            pltpu.VMEM_SHARED((_NSP, 16), jnp.float32),
            pltpu.VMEM_SHARED((_NSP,), jnp.float32),
        ],
    )
    return kfn(src3, dst3, as4, ad4, zflat, mrep)


def _post_body(acc_ref, s_ref, b_ref, s3_ref, b3_ref,
               w1_ref, b1_ref, w2_ref, b2_ref,
               out_ref, pooled, cnt):
    i = pl.program_id(0)
    s = s_ref[...] + jnp.float32(1e-16)
    inv = 1.0 / s
    parts = []
    for hd in range(_H):
        vh = jnp.concatenate([acc_ref[hd, 0], acc_ref[hd, 1]], axis=1)
        parts.append(vh * inv[:, hd:hd + 1])
    v = jnp.concatenate(parts, axis=1)
    v = v * s3_ref[...] + b3_ref[...]
    v = jnp.where(v > 0.0, v, jnp.exp(v) - 1.0)

    g_iota = lax.broadcasted_iota(jnp.int32, (1, _G), 1)
    oh = (b_ref[...] == g_iota).astype(jnp.float32)
    dn = (((0,), (0,)), ((), ()))
    pool_inc = lax.dot_general(oh, v, dn, preferred_element_type=jnp.float32)
    cnt_inc = lax.dot_general(oh, jnp.ones((_R, 8), jnp.float32), dn,
                              preferred_element_type=jnp.float32)

    @pl.when(i == 0)
    def _():
        pooled[...] = pool_inc
        cnt[...] = cnt_inc

    @pl.when(i != 0)
    def _():
        pooled[...] = pooled[...] + pool_inc
        cnt[...] = cnt[...] + cnt_inc

    @pl.when(i == _GRID - 1)
    def _():
        mean = pooled[...] / jnp.maximum(cnt[...][:, :1], 1.0)
        t = jnp.dot(mean, w1_ref[...], preferred_element_type=jnp.float32)
        t = jnp.maximum(t + b1_ref[...], 0.0)
        out = jnp.dot(t, w2_ref[...], preferred_element_type=jnp.float32)
        out_ref[...] = out + b2_ref[...]


def _post_call(accumT, s2, batch2, s3, b3, w1, b1, w2, b2):
    full = lambda shape: pl.BlockSpec(shape, lambda i: (0,) * len(shape))
    return pl.pallas_call(
        _post_body,
        grid=(_GRID,),
        in_specs=[
            pl.BlockSpec((_H, 2, _R, 16), lambda i: (0, 0, i, 0)),
            pl.BlockSpec((_R, _H), lambda i: (i, 0)),
            pl.BlockSpec((_R, 1), lambda i: (i, 0)),
            full((1, _H * _C)), full((1, _H * _C)),
            full((_H * _C, 32)), full((1, 32)), full((32, 2)), full((1, 2)),
        ],
        out_specs=pl.BlockSpec((_G, 2), lambda i: (0, 0)),
        out_shape=jax.ShapeDtypeStruct((_G, 2), jnp.float32),
        scratch_shapes=[
            pltpu.VMEM((_G, _H * _C), jnp.float32),
            pltpu.VMEM((_G, 8), jnp.float32),
        ],
    )(accumT, s2, batch2, s3, b3, w1, b1, w2, b2)


def kernel(x, edge_index, batch, ln_g, ln_b, bn1_g, bn1_b, bn1_m, bn1_v,
           cnn_W, cnn_b, bn2_g, bn2_b, bn2_m, bn2_v, pre_W, pre_b, gat_W,
           att_src, att_dst, gat_bias, bn3_g, bn3_b, bn3_m, bn3_v,
           mlp_W1, mlp_b1, mlp_W2, mlp_b2):
    f32 = jnp.float32
    s1 = bn1_g / jnp.sqrt(bn1_v + _EPS)
    b1 = bn1_b - bn1_m * s1
    c16_W = s1[:, None] * cnn_W
    c16_b = b1 @ cnn_W + cnn_b
    s2 = bn2_g / jnp.sqrt(bn2_v + _EPS)
    b2 = bn2_b - bn2_m * s2
    w_a = (c16_W * s2[None, :16]) @ pre_W[:16]
    w_b = (ln_g * s2[16:])[:, None] * pre_W[16:]
    b_ab = ((c16_b * s2[:16] + b2[:16]) @ pre_W[:16]
            + (ln_b * s2[16:] + b2[16:]) @ pre_W[16:] + pre_b)
    as_w = jnp.einsum("khc,hc->kh", gat_W.reshape(32, _H, _C), att_src)
    ad_w = jnp.einsum("khc,hc->kh", gat_W.reshape(32, _H, _C), att_dst)
    as_w = jnp.pad(as_w, ((0, 0), (0, 16 - _H)))
    ad_w = jnp.pad(ad_w, ((0, 0), (0, 16 - _H)))

    zT, as4, ad4, M = _pre_call(x, w_a, w_b, b_ab.reshape(1, 32), gat_W,
                                as_w, ad_w)

    t = M[0, :_H] + M[0, _H:]
    shift = jnp.where(t < 0, 0.2 * t, t)
    mrep = jnp.tile(shift, 4).astype(f32)

    ereal = edge_index.shape[1] + _N
    pad = _EPAD - ereal
    loops = jnp.arange(_N, dtype=jnp.int32)
    zpad = jnp.zeros((pad,), jnp.int32)
    src3 = jnp.concatenate([edge_index[0], loops, zpad]).reshape(_NCH, 8, 128)
    dst3 = jnp.concatenate([edge_index[1], loops, zpad]).reshape(_NCH, 8, 128)
    zflat = zT.reshape(2 * _H * _N, 16)

    accumT, s2p, _spill = _edge_call(ereal, src3, dst3, as4, ad4, zflat, mrep)
    s4 = s2p.reshape(_H, _NSP).T

    s3 = bn3_g / jnp.sqrt(bn3_v + _EPS)
    b3 = bn3_b + (gat_bias - bn3_m) * s3
    return _post_call(accumT, s4, batch.reshape(_N, 1),
                      s3.reshape(1, _H * _C), b3.reshape(1, _H * _C),
                      mlp_W1, mlp_b1.reshape(1, 32), mlp_W2,
                      mlp_b2.reshape(1, 2))


# speedup vs baseline: 27.8010x; 1.0536x over previous
import functools

import jax
import jax.numpy as jnp
from jax import lax
from jax.experimental import pallas as pl
from jax.experimental.pallas import tpu as pltpu
from jax.experimental.pallas import tpu_sc as plsc

_N = 50000
_H = 4
_C = 32
_G = 256
_EPS = 1e-5
_R = 400
_GRID = _N // _R

_NSC = 2
_NTILE = 16
_CHUNK = 1024
_CPT = 52
_NCH = _NTILE * _CPT
_EPAD = _NCH * _CHUNK
_RPT = 3128
_NSP = _NTILE * _RPT


def _pre_body(x_ref, wa_ref, wb_ref, b_ref, gw_ref, as_w_ref, ad_w_ref,
              z_ref, as_ref, ad_ref, m_ref):
    i = pl.program_id(0)
    xb = x_ref[...]
    xc = xb[:, :512]
    xf = xb[:, 512:]
    mu = jnp.mean(xf, axis=-1, keepdims=True)
    var = jnp.mean((xf - mu) ** 2, axis=-1, keepdims=True)
    xfn = (xf - mu) / jnp.sqrt(var + _EPS)
    h = jnp.dot(xc, wa_ref[...], preferred_element_type=jnp.float32)
    h = h + jnp.dot(xfn, wb_ref[...], preferred_element_type=jnp.float32)
    h = jnp.maximum(h + b_ref[...], 0.0)
    z = jnp.dot(h, gw_ref[...], preferred_element_type=jnp.float32)
    for hd in range(_H):
        z_ref[hd] = z[:, hd * _C:(hd + 1) * _C]
    a_s = jnp.dot(h, as_w_ref[...], preferred_element_type=jnp.float32)
    a_d = jnp.dot(h, ad_w_ref[...], preferred_element_type=jnp.float32)
    as_ref[...] = a_s
    ad_ref[...] = a_d
    mx = jnp.concatenate([jnp.max(a_s, axis=0)[:_H], jnp.max(a_d, axis=0)[:_H]])
    mx = mx.reshape(1, 2 * _H)

    @pl.when(i == 0)
    def _():
        m_ref[...] = mx

    @pl.when(i != 0)
    def _():
        m_ref[...] = jnp.maximum(m_ref[...], mx)


def _pre_call(x, w_a, w_b, b_ab, gat_W, as_w, ad_w):
    full = lambda shape: pl.BlockSpec(shape, lambda i: (0,) * len(shape))
    return pl.pallas_call(
        _pre_body,
        grid=(_GRID,),
        in_specs=[
            pl.BlockSpec((_R, 576), lambda i: (i, 0)),
            full((512, 32)), full((64, 32)), full((1, 32)),
            full((32, _H * _C)), full((32, 16)), full((32, 16)),
        ],
        out_specs=[
            pl.BlockSpec((_H, _R, _C), lambda i: (0, i, 0)),
            pl.BlockSpec((_R, 16), lambda i: (i, 0)),
            pl.BlockSpec((_R, 16), lambda i: (i, 0)),
            pl.BlockSpec((1, 2 * _H), lambda i: (0, 0)),
        ],
        out_shape=[
            jax.ShapeDtypeStruct((_H, _N, _C), jnp.float32),
            jax.ShapeDtypeStruct((_N, 16), jnp.float32),
            jax.ShapeDtypeStruct((_N, 16), jnp.float32),
            jax.ShapeDtypeStruct((1, 2 * _H), jnp.float32),
        ],
    )(x, w_a, w_b, b_ab, gat_W, as_w, ad_w)


def _edge_body(ereal, src_h, dst_h, as_h, ad_h, zf_h, mrep_h,
               accum_o, s_o, exsp_o,
               src_v, dst_v, srcz_v, as_v, ad_v, exh_v, exh1_v,
               zs_v, zrows_v, zbuf_v, mrep_v, sem,
               accum_sp, s_sp):
    cid = lax.axis_index("c")
    sid = lax.axis_index("s")
    iota = lax.iota(jnp.int32, 16)
    zeros16 = jnp.zeros((16,), jnp.float32)

    def _z0(e, _):
        zbuf_v[e, pl.ds(0, 16)] = zeros16
        return 0
    lax.fori_loop(0, _CHUNK, _z0, 0)

    def _z1(j, _):
        zs_v[pl.ds(j * 16, 16)] = zeros16
        return 0
    lax.fori_loop(0, _CHUNK // 16, _z1, 0)

    pltpu.sync_copy(mrep_h, mrep_v)

    r0 = sid * _RPT

    def zero_accum():
        for q in range(3):
            pltpu.sync_copy(zbuf_v, accum_sp.at[pl.ds(r0 + q * 1024, 1024), :])
        pltpu.sync_copy(zbuf_v.at[pl.ds(0, 56), :],
                        accum_sp.at[pl.ds(r0 + 3072, 56), :])

    def zero_s():
        for q in range(3):
            pltpu.sync_copy(zs_v, s_sp.at[pl.ds(r0 + q * 1024, 1024)])
        pltpu.sync_copy(zs_v.at[pl.ds(0, 56)], s_sp.at[pl.ds(r0 + 3072, 56)])

    zero_accum()
    zero_s()
    plsc.subcore_barrier()

    mrv = mrep_v[...]

    for p in range(4):
        hp, ch = p >> 1, p & 1
        head = 2 * cid + hp
        zmul_off = 2 * head * _N + ch

        def chunk_body(k, _, p=p, head=head, zmul_off=zmul_off):
            cg = sid * _CPT + k
            pltpu.sync_copy(src_h.at[cg], src_v)
            pltpu.sync_copy(dst_h.at[cg], dst_v)

            if p == 0:
                descs = []
                for j in range(8):
                    descs.append(pltpu.async_copy(
                        as_h.at[src_v.at[j]],
                        as_v.at[pl.ds(j * 128, 128), :], sem))
                    descs.append(pltpu.async_copy(
                        ad_h.at[dst_v.at[j]],
                        ad_v.at[pl.ds(j * 128, 128), :], sem))
                for d in descs:
                    d.wait()

                hm0 = iota == head
                hm1 = iota == (head + 1)

                cbase = cg * _CHUNK

                def _cex(i, _):
                    for t in range(4):
                        e = i * 4 + t
                        a16 = as_v[e, pl.ds(0, 16)]
                        d16 = ad_v[e, pl.ds(0, 16)]
                        u = a16 + d16
                        u = jnp.where(u < 0.0, u * jnp.float32(0.2), u)
                        ex = jnp.exp(u - mrv)
                        eidx = jnp.broadcast_to(e, (16,))
                        ok = (cbase + eidx) < ereal
                        ex = jnp.where(ok, ex, 0.0)
                        plsc.store_scatter(exh_v, [eidx], ex, mask=hm0)
                        plsc.store_scatter(exh1_v, [eidx], ex, mask=hm1)
                    return 0
                lax.fori_loop(0, _CHUNK // 4, _cex, 0)
                pltpu.sync_copy(exh_v,
                                exsp_o.at[pl.ds((head * _NCH + cg) * _CHUNK,
                                                _CHUNK)])
                pltpu.sync_copy(exh1_v,
                                exsp_o.at[pl.ds(((head + 1) * _NCH + cg)
                                                * _CHUNK, _CHUNK)])
            else:
                pltpu.sync_copy(exsp_o.at[pl.ds((head * _NCH + cg) * _CHUNK,
                                                _CHUNK)], exh_v)

            def _off(j3, _):
                sv = src_v[j3 >> 3, pl.ds((j3 & 7) * 16, 16)]
                srcz_v[j3 >> 3, pl.ds((j3 & 7) * 16, 16)] = sv * 2 + zmul_off
                return 0
            lax.fori_loop(0, _CHUNK // 16, _off, 0)

            descs = [pltpu.async_copy(zf_h.at[srcz_v.at[j]],
                                      zrows_v.at[pl.ds(j * 128, 128), :], sem)
                     for j in range(8)]
            for d in descs:
                d.wait()

            def _scale(i, _):
                for t in range(8):
                    e = i * 8 + t
                    exb = plsc.load_gather(exh_v, [jnp.broadcast_to(e, (16,))])
                    zrows_v[e, pl.ds(0, 16)] = zrows_v[e, pl.ds(0, 16)] * exb
                return 0
            lax.fori_loop(0, _CHUNK // 8, _scale, 0)

            descs = []
            for j in range(8):
                descs.append(pltpu.async_copy(
                    zrows_v.at[pl.ds(j * 128, 128), :],
                    accum_sp.at[dst_v.at[j]], sem, add=True))
                if p == 0 or p == 2:
                    descs.append(pltpu.async_copy(
                        exh_v.at[pl.ds(j * 128, 128)],
                        s_sp.at[dst_v.at[j]], sem, add=True))
            for d in descs:
                d.wait()
            return 0

        lax.fori_loop(0, _CPT, chunk_body, 0)
        plsc.subcore_barrier()

        for q in range(3):
            pltpu.sync_copy(accum_sp.at[pl.ds(r0 + q * 1024, 1024), :],
                            accum_o.at[head, ch, pl.ds(r0 + q * 1024, 1024), :])
        pltpu.sync_copy(accum_sp.at[pl.ds(r0 + 3072, 56), :],
                        accum_o.at[head, ch, pl.ds(r0 + 3072, 56), :])
        if p == 0 or p == 2:
            sbase = head * _NSP + r0
            for q in range(3):
                pltpu.sync_copy(s_sp.at[pl.ds(r0 + q * 1024, 1024)],
                                s_o.at[pl.ds(sbase + q * 1024, 1024)])
            pltpu.sync_copy(s_sp.at[pl.ds(r0 + 3072, 56)],
                            s_o.at[pl.ds(sbase + 3072, 56)])
        if p != 3:
            zero_accum()
            if p == 1:
                zero_s()
            plsc.subcore_barrier()


def _edge_call(ereal, src3, dst3, as4, ad4, zflat, mrep):
    mesh = plsc.VectorSubcoreMesh(core_axis_name="c", subcore_axis_name="s",
                                  num_cores=_NSC, num_subcores=_NTILE)
    kfn = pl.kernel(
        functools.partial(_edge_body, ereal),
        compiler_params=pltpu.CompilerParams(needs_layout_passes=False,
                                             use_tc_tiling_on_sc=False),
        out_type=(
            jax.ShapeDtypeStruct((_H, 2, _NSP, 16), jnp.float32),
            jax.ShapeDtypeStruct((_H * _NSP,), jnp.float32),
            jax.ShapeDtypeStruct((_H * _NCH * _CHUNK,), jnp.float32),
        ),
        mesh=mesh,
        scratch_types=[
            pltpu.VMEM((8, 128), jnp.int32),
            pltpu.VMEM((8, 128), jnp.int32),
            pltpu.VMEM((8, 128), jnp.int32),
            pltpu.VMEM((_CHUNK, 16), jnp.float32),
            pltpu.VMEM((_CHUNK, 16), jnp.float32),
            pltpu.VMEM((_CHUNK,), jnp.float32),
            pltpu.VMEM((_CHUNK,), jnp.float32),
            pltpu.VMEM((_CHUNK,), jnp.float32),
            pltpu.VMEM((_CHUNK, 16), jnp.float32),
            pltpu.VMEM((_CHUNK, 16), jnp.float32),
            pltpu.VMEM((16,), jnp.float32),
            pltpu.SemaphoreType.DMA,
            pltpu.VMEM_SHARED((_NSP, 16), jnp.float32),
            pltpu.VMEM_SHARED((_NSP,), jnp.float32),
        ],
    )
    return kfn(src3, dst3, as4, ad4, zflat, mrep)


def _post_body(acc_ref, s_ref, b_ref, s3_ref, b3_ref,
               w1_ref, b1_ref, w2_ref, b2_ref,
               out_ref, pooled, cnt):
    i = pl.program_id(0)
    s = s_ref[...] + jnp.float32(1e-16)
    inv = 1.0 / s
    parts = []
    for hd in range(_H):
        vh = jnp.concatenate([acc_ref[hd, 0], acc_ref[hd, 1]], axis=1)
        parts.append(vh * inv[:, hd:hd + 1])
    v = jnp.concatenate(parts, axis=1)
    v = v * s3_ref[...] + b3_ref[...]
    v = jnp.where(v > 0.0, v, jnp.exp(v) - 1.0)

    g_iota = lax.broadcasted_iota(jnp.int32, (1, _G), 1)
    oh = (b_ref[...] == g_iota).astype(jnp.float32)
    dn = (((0,), (0,)), ((), ()))
    pool_inc = lax.dot_general(oh, v, dn, preferred_element_type=jnp.float32)
    cnt_inc = lax.dot_general(oh, jnp.ones((_R, 8), jnp.float32), dn,
                              preferred_element_type=jnp.float32)

    @pl.when(i == 0)
    def _():
        pooled[...] = pool_inc
        cnt[...] = cnt_inc

    @pl.when(i != 0)
    def _():
        pooled[...] = pooled[...] + pool_inc
        cnt[...] = cnt[...] + cnt_inc

    @pl.when(i == _GRID - 1)
    def _():
        mean = pooled[...] / jnp.maximum(cnt[...][:, :1], 1.0)
        t = jnp.dot(mean, w1_ref[...], preferred_element_type=jnp.float32)
        t = jnp.maximum(t + b1_ref[...], 0.0)
        out = jnp.dot(t, w2_ref[...], preferred_element_type=jnp.float32)
        out_ref[...] = out + b2_ref[...]


def _post_call(accumT, s2, batch2, s3, b3, w1, b1, w2, b2):
    full = lambda shape: pl.BlockSpec(shape, lambda i: (0,) * len(shape))
    return pl.pallas_call(
        _post_body,
        grid=(_GRID,),
        in_specs=[
            pl.BlockSpec((_H, 2, _R, 16), lambda i: (0, 0, i, 0)),
            pl.BlockSpec((_R, _H), lambda i: (i, 0)),
            pl.BlockSpec((_R, 1), lambda i: (i, 0)),
            full((1, _H * _C)), full((1, _H * _C)),
            full((_H * _C, 32)), full((1, 32)), full((32, 2)), full((1, 2)),
        ],
        out_specs=pl.BlockSpec((_G, 2), lambda i: (0, 0)),
        out_shape=jax.ShapeDtypeStruct((_G, 2), jnp.float32),
        scratch_shapes=[
            pltpu.VMEM((_G, _H * _C), jnp.float32),
            pltpu.VMEM((_G, 8), jnp.float32),
        ],
    )(accumT, s2, batch2, s3, b3, w1, b1, w2, b2)


def kernel(x, edge_index, batch, ln_g, ln_b, bn1_g, bn1_b, bn1_m, bn1_v,
           cnn_W, cnn_b, bn2_g, bn2_b, bn2_m, bn2_v, pre_W, pre_b, gat_W,
           att_src, att_dst, gat_bias, bn3_g, bn3_b, bn3_m, bn3_v,
           mlp_W1, mlp_b1, mlp_W2, mlp_b2):
    f32 = jnp.float32
    s1 = bn1_g / jnp.sqrt(bn1_v + _EPS)
    b1 = bn1_b - bn1_m * s1
    c16_W = s1[:, None] * cnn_W
    c16_b = b1 @ cnn_W + cnn_b
    s2 = bn2_g / jnp.sqrt(bn2_v + _EPS)
    b2 = bn2_b - bn2_m * s2
    w_a = (c16_W * s2[None, :16]) @ pre_W[:16]
    w_b = (ln_g * s2[16:])[:, None] * pre_W[16:]
    b_ab = ((c16_b * s2[:16] + b2[:16]) @ pre_W[:16]
            + (ln_b * s2[16:] + b2[16:]) @ pre_W[16:] + pre_b)
    as_w = jnp.einsum("khc,hc->kh", gat_W.reshape(32, _H, _C), att_src)
    ad_w = jnp.einsum("khc,hc->kh", gat_W.reshape(32, _H, _C), att_dst)
    as_w = jnp.pad(as_w, ((0, 0), (0, 16 - _H)))
    ad_w = jnp.pad(ad_w, ((0, 0), (0, 16 - _H)))

    zT, as4, ad4, M = _pre_call(x, w_a, w_b, b_ab.reshape(1, 32), gat_W,
                                as_w, ad_w)

    t = M[0, :_H] + M[0, _H:]
    shift = jnp.where(t < 0, 0.2 * t, t)
    mrep = jnp.tile(shift, 4).astype(f32)

    ereal = edge_index.shape[1] + _N
    pad = _EPAD - ereal
    loops = jnp.arange(_N, dtype=jnp.int32)
    zpad = jnp.zeros((pad,), jnp.int32)
    src3 = jnp.concatenate([edge_index[0], loops, zpad]).reshape(_NCH, 8, 128)
    dst3 = jnp.concatenate([edge_index[1], loops, zpad]).reshape(_NCH, 8, 128)
    zflat = zT.reshape(2 * _H * _N, 16)

    accumT, s2p, _spill = _edge_call(ereal, src3, dst3, as4, ad4, zflat, mrep)
    s4 = s2p.reshape(_H, _NSP).T

    s3 = bn3_g / jnp.sqrt(bn3_v + _EPS)
    b3 = bn3_b + (gat_bias - bn3_m) * s3
    return _post_call(accumT, s4, batch.reshape(_N, 1),
                      s3.reshape(1, _H * _C), b3.reshape(1, _H * _C),
                      mlp_W1, mlp_b1.reshape(1, 32), mlp_W2,
                      mlp_b2.reshape(1, 2))


# speedup vs baseline: 31.4356x; 1.1307x over previous
import functools

import jax
import jax.numpy as jnp
from jax import lax
from jax.experimental import pallas as pl
from jax.experimental.pallas import tpu as pltpu
from jax.experimental.pallas import tpu_sc as plsc

_N = 50000
_H = 4
_C = 32
_G = 256
_EPS = 1e-5
_R = 400
_GRID = _N // _R

_NSC = 2
_NTILE = 16
_CHUNK = 1024
_CPT = 52
_NCH = _NTILE * _CPT
_EPAD = _NCH * _CHUNK
_RPT = 3128
_NSP = _NTILE * _RPT


def _pre_body(x_ref, wa_ref, wb_ref, b_ref, gw_ref, as_w_ref, ad_w_ref,
              z_ref, as_ref, ad_ref, m_ref):
    i = pl.program_id(0)
    xb = x_ref[...]
    xc = xb[:, :512]
    xf = xb[:, 512:]
    mu = jnp.mean(xf, axis=-1, keepdims=True)
    var = jnp.mean((xf - mu) ** 2, axis=-1, keepdims=True)
    xfn = (xf - mu) / jnp.sqrt(var + _EPS)
    h = jnp.dot(xc, wa_ref[...], preferred_element_type=jnp.float32)
    h = h + jnp.dot(xfn, wb_ref[...], preferred_element_type=jnp.float32)
    h = jnp.maximum(h + b_ref[...], 0.0)
    z = jnp.dot(h, gw_ref[...], preferred_element_type=jnp.float32)
    for hd in range(_H):
        z_ref[hd] = z[:, hd * _C:(hd + 1) * _C]
    a_s = jnp.dot(h, as_w_ref[...], preferred_element_type=jnp.float32)
    a_d = jnp.dot(h, ad_w_ref[...], preferred_element_type=jnp.float32)
    as_ref[...] = a_s
    ad_ref[...] = a_d
    mx = jnp.concatenate([jnp.max(a_s, axis=0)[:_H], jnp.max(a_d, axis=0)[:_H]])
    mx = mx.reshape(1, 2 * _H)

    @pl.when(i == 0)
    def _():
        m_ref[...] = mx

    @pl.when(i != 0)
    def _():
        m_ref[...] = jnp.maximum(m_ref[...], mx)


def _pre_call(x, w_a, w_b, b_ab, gat_W, as_w, ad_w):
    full = lambda shape: pl.BlockSpec(shape, lambda i: (0,) * len(shape))
    return pl.pallas_call(
        _pre_body,
        grid=(_GRID,),
        in_specs=[
            pl.BlockSpec((_R, 576), lambda i: (i, 0)),
            full((512, 32)), full((64, 32)), full((1, 32)),
            full((32, _H * _C)), full((32, 16)), full((32, 16)),
        ],
        out_specs=[
            pl.BlockSpec((_H, _R, _C), lambda i: (0, i, 0)),
            pl.BlockSpec((_R, 16), lambda i: (i, 0)),
            pl.BlockSpec((_R, 16), lambda i: (i, 0)),
            pl.BlockSpec((1, 2 * _H), lambda i: (0, 0)),
        ],
        out_shape=[
            jax.ShapeDtypeStruct((_H, _N, _C), jnp.float32),
            jax.ShapeDtypeStruct((_N, 16), jnp.float32),
            jax.ShapeDtypeStruct((_N, 16), jnp.float32),
            jax.ShapeDtypeStruct((1, 2 * _H), jnp.float32),
        ],
    )(x, w_a, w_b, b_ab, gat_W, as_w, ad_w)


def _edge_body(ereal, src_h, dst_h, as_h, ad_h, zf_h, mrep_h,
               accum_o, s_o, exsp_o,
               src_v, dst_v, srcz_v, src2_v, dst2_v, srcz2_v,
               exh_v, exh1_v,
               zs_v, zrows_v, zrows2_v, zbuf_v, mrep_v, sem, semz, sems,
               accum_sp, s_sp):
    as_v = zrows2_v
    ad_v = zbuf_v
    cid = lax.axis_index("c")
    sid = lax.axis_index("s")
    iota = lax.iota(jnp.int32, 16)
    zeros16 = jnp.zeros((16,), jnp.float32)

    def _z0(e, _):
        zbuf_v[e, pl.ds(0, 16)] = zeros16
        return 0

    def _z1(j, _):
        zs_v[pl.ds(j * 16, 16)] = zeros16
        return 0
    lax.fori_loop(0, _CHUNK // 16, _z1, 0)

    pltpu.sync_copy(mrep_h, mrep_v)

    r0 = sid * _RPT

    def zero_accum():
        lax.fori_loop(0, _CHUNK, _z0, 0)
        for q in range(3):
            pltpu.sync_copy(zbuf_v, accum_sp.at[pl.ds(r0 + q * 1024, 1024), :])
        pltpu.sync_copy(zbuf_v.at[pl.ds(0, 56), :],
                        accum_sp.at[pl.ds(r0 + 3072, 56), :])

    def zero_s():
        for q in range(3):
            pltpu.sync_copy(zs_v, s_sp.at[pl.ds(r0 + q * 1024, 1024)])
        pltpu.sync_copy(zs_v.at[pl.ds(0, 56)], s_sp.at[pl.ds(r0 + 3072, 56)])

    zero_accum()
    zero_s()
    plsc.subcore_barrier()

    mrv = mrep_v[...]

    srcB = [src_v, src2_v]
    dstB = [dst_v, dst2_v]
    srczB = [srcz_v, srcz2_v]
    zrowsB = [zrows_v, zrows2_v]
    exhB = [exh_v, exh1_v]

    def _mk_off(srcb, srczb, zmul_off):
        def _off(j3, _):
            sv = srcb[j3 >> 3, pl.ds((j3 & 7) * 16, 16)]
            srczb[j3 >> 3, pl.ds((j3 & 7) * 16, 16)] = sv * 2 + zmul_off
            return 0
        return _off

    def _mk_scale(zrowsb, exhb):
        def _scale(i, _):
            for t in range(8):
                e = i * 8 + t
                exb = plsc.load_gather(exhb, [jnp.broadcast_to(e, (16,))])
                zrowsb[e, pl.ds(0, 16)] = zrowsb[e, pl.ds(0, 16)] * exb
            return 0
        return _scale

    for p in range(4):
        hp, ch = p >> 1, p & 1
        head = 2 * cid + hp
        zmul_off = 2 * head * _N + ch

        if p == 0:
            hm0 = iota == head
            hm1 = iota == (head + 1)

            def chunk_body(k, _, head=head, zmul_off=zmul_off,
                           hm0=hm0, hm1=hm1):
                cg = sid * _CPT + k
                pltpu.sync_copy(src_h.at[cg], src_v)
                pltpu.sync_copy(dst_h.at[cg], dst_v)

                lax.fori_loop(0, _CHUNK // 16,
                              _mk_off(src_v, srcz_v, zmul_off), 0)
                zdescs = [pltpu.async_copy(
                    zf_h.at[srcz_v.at[j]],
                    zrows_v.at[pl.ds(j * 128, 128), :], semz)
                    for j in range(8)]

                descs = []
                for j in range(8):
                    descs.append(pltpu.async_copy(
                        as_h.at[src_v.at[j]],
                        as_v.at[pl.ds(j * 128, 128), :], sem))
                    descs.append(pltpu.async_copy(
                        ad_h.at[dst_v.at[j]],
                        ad_v.at[pl.ds(j * 128, 128), :], sem))
                for d in descs:
                    d.wait()

                cbase = cg * _CHUNK

                def _cex(i, _):
                    for t in range(4):
                        e = i * 4 + t
                        a16 = as_v[e, pl.ds(0, 16)]
                        d16 = ad_v[e, pl.ds(0, 16)]
                        u = a16 + d16
                        u = jnp.where(u < 0.0, u * jnp.float32(0.2), u)
                        ex = jnp.exp(u - mrv)
                        eidx = jnp.broadcast_to(e, (16,))
                        ok = (cbase + eidx) < ereal
                        ex = jnp.where(ok, ex, 0.0)
                        plsc.store_scatter(exh_v, [eidx], ex, mask=hm0)
                        plsc.store_scatter(exh1_v, [eidx], ex, mask=hm1)
                    return 0
                lax.fori_loop(0, _CHUNK // 4, _cex, 0)
                pltpu.sync_copy(exh_v,
                                exsp_o.at[pl.ds((head * _NCH + cg) * _CHUNK,
                                                _CHUNK)])
                pltpu.sync_copy(exh1_v,
                                exsp_o.at[pl.ds(((head + 1) * _NCH + cg)
                                                * _CHUNK, _CHUNK)])

                for d in zdescs:
                    d.wait()
                lax.fori_loop(0, _CHUNK // 8, _mk_scale(zrows_v, exh_v), 0)

                descs = []
                for j in range(8):
                    descs.append(pltpu.async_copy(
                        zrows_v.at[pl.ds(j * 128, 128), :],
                        accum_sp.at[dst_v.at[j]], sems, add=True))
                    descs.append(pltpu.async_copy(
                        exh_v.at[pl.ds(j * 128, 128)],
                        s_sp.at[dst_v.at[j]], sems, add=True))
                for d in descs:
                    d.wait()
                return 0

            lax.fori_loop(0, _CPT, chunk_body, 0)
        else:
            def pair_body(i, _, p=p, head=head, zmul_off=zmul_off):
                lds = [None, None]
                for b in range(2):
                    cg = sid * _CPT + i * 2 + b
                    lds[b] = ([pltpu.async_copy(src_h.at[cg], srcB[b], sem),
                               pltpu.async_copy(dst_h.at[cg], dstB[b], sem),
                               pltpu.async_copy(
                                   exsp_o.at[pl.ds((head * _NCH + cg)
                                                   * _CHUNK, _CHUNK)],
                                   exhB[b], sem)], cg)
                zds = [None, None]
                for b in range(2):
                    for d in lds[b][0]:
                        d.wait()
                    lax.fori_loop(0, _CHUNK // 16,
                                  _mk_off(srcB[b], srczB[b], zmul_off), 0)
                    zds[b] = [pltpu.async_copy(
                        zf_h.at[srczB[b].at[j]],
                        zrowsB[b].at[pl.ds(j * 128, 128), :], semz)
                        for j in range(8)]
                sds = [None, None]
                for b in range(2):
                    for d in zds[b]:
                        d.wait()
                    lax.fori_loop(0, _CHUNK // 8,
                                  _mk_scale(zrowsB[b], exhB[b]), 0)
                    descs = []
                    for j in range(8):
                        descs.append(pltpu.async_copy(
                            zrowsB[b].at[pl.ds(j * 128, 128), :],
                            accum_sp.at[dstB[b].at[j]], sems, add=True))
                        if p == 2:
                            descs.append(pltpu.async_copy(
                                exhB[b].at[pl.ds(j * 128, 128)],
                                s_sp.at[dstB[b].at[j]], sems, add=True))
                    sds[b] = descs
                for b in range(2):
                    for d in sds[b]:
                        d.wait()
                return 0

            lax.fori_loop(0, _CPT // 2, pair_body, 0)

        plsc.subcore_barrier()

        for q in range(3):
            pltpu.sync_copy(accum_sp.at[pl.ds(r0 + q * 1024, 1024), :],
                            accum_o.at[head, ch, pl.ds(r0 + q * 1024, 1024), :])
        pltpu.sync_copy(accum_sp.at[pl.ds(r0 + 3072, 56), :],
                        accum_o.at[head, ch, pl.ds(r0 + 3072, 56), :])
        if p == 0 or p == 2:
            sbase = head * _NSP + r0
            for q in range(3):
                pltpu.sync_copy(s_sp.at[pl.ds(r0 + q * 1024, 1024)],
                                s_o.at[pl.ds(sbase + q * 1024, 1024)])
            pltpu.sync_copy(s_sp.at[pl.ds(r0 + 3072, 56)],
                            s_o.at[pl.ds(sbase + 3072, 56)])
        if p != 3:
            zero_accum()
            if p == 1:
                zero_s()
            plsc.subcore_barrier()


def _edge_call(ereal, src3, dst3, as4, ad4, zflat, mrep):
    mesh = plsc.VectorSubcoreMesh(core_axis_name="c", subcore_axis_name="s",
                                  num_cores=_NSC, num_subcores=_NTILE)
    kfn = pl.kernel(
        functools.partial(_edge_body, ereal),
        compiler_params=pltpu.CompilerParams(needs_layout_passes=False,
                                             use_tc_tiling_on_sc=False),
        out_type=(
            jax.ShapeDtypeStruct((_H, 2, _NSP, 16), jnp.float32),
            jax.ShapeDtypeStruct((_H * _NSP,), jnp.float32),
            jax.ShapeDtypeStruct((_H * _NCH * _CHUNK,), jnp.float32),
        ),
        mesh=mesh,
        scratch_types=[
            pltpu.VMEM((8, 128), jnp.int32),
            pltpu.VMEM((8, 128), jnp.int32),
            pltpu.VMEM((8, 128), jnp.int32),
            pltpu.VMEM((8, 128), jnp.int32),
            pltpu.VMEM((8, 128), jnp.int32),
            pltpu.VMEM((8, 128), jnp.int32),
            pltpu.VMEM((_CHUNK,), jnp.float32),
            pltpu.VMEM((_CHUNK,), jnp.float32),
            pltpu.VMEM((_CHUNK,), jnp.float32),
            pltpu.VMEM((_CHUNK, 16), jnp.float32),
            pltpu.VMEM((_CHUNK, 16), jnp.float32),
            pltpu.VMEM((_CHUNK, 16), jnp.float32),
            pltpu.VMEM((16,), jnp.float32),
            pltpu.SemaphoreType.DMA,
            pltpu.SemaphoreType.DMA,
            pltpu.SemaphoreType.DMA,
            pltpu.VMEM_SHARED((_NSP, 16), jnp.float32),
            pltpu.VMEM_SHARED((_NSP,), jnp.float32),
        ],
    )
    return kfn(src3, dst3, as4, ad4, zflat, mrep)


def _post_body(acc_ref, s_ref, b_ref, s3_ref, b3_ref,
               w1_ref, b1_ref, w2_ref, b2_ref,
               out_ref, pooled, cnt):
    i = pl.program_id(0)
    s = s_ref[...] + jnp.float32(1e-16)
    inv = 1.0 / s
    parts = []
    for hd in range(_H):
        vh = jnp.concatenate([acc_ref[hd, 0], acc_ref[hd, 1]], axis=1)
        parts.append(vh * inv[:, hd:hd + 1])
    v = jnp.concatenate(parts, axis=1)
    v = v * s3_ref[...] + b3_ref[...]
    v = jnp.where(v > 0.0, v, jnp.exp(v) - 1.0)

    g_iota = lax.broadcasted_iota(jnp.int32, (1, _G), 1)
    oh = (b_ref[...] == g_iota).astype(jnp.float32)
    dn = (((0,), (0,)), ((), ()))
    pool_inc = lax.dot_general(oh, v, dn, preferred_element_type=jnp.float32)
    cnt_inc = lax.dot_general(oh, jnp.ones((_R, 8), jnp.float32), dn,
                              preferred_element_type=jnp.float32)

    @pl.when(i == 0)
    def _():
        pooled[...] = pool_inc
        cnt[...] = cnt_inc

    @pl.when(i != 0)
    def _():
        pooled[...] = pooled[...] + pool_inc
        cnt[...] = cnt[...] + cnt_inc

    @pl.when(i == _GRID - 1)
    def _():
        mean = pooled[...] / jnp.maximum(cnt[...][:, :1], 1.0)
        t = jnp.dot(mean, w1_ref[...], preferred_element_type=jnp.float32)
        t = jnp.maximum(t + b1_ref[...], 0.0)
        out = jnp.dot(t, w2_ref[...], preferred_element_type=jnp.float32)
        out_ref[...] = out + b2_ref[...]


def _post_call(accumT, s2, batch2, s3, b3, w1, b1, w2, b2):
    full = lambda shape: pl.BlockSpec(shape, lambda i: (0,) * len(shape))
    return pl.pallas_call(
        _post_body,
        grid=(_GRID,),
        in_specs=[
            pl.BlockSpec((_H, 2, _R, 16), lambda i: (0, 0, i, 0)),
            pl.BlockSpec((_R, _H), lambda i: (i, 0)),
            pl.BlockSpec((_R, 1), lambda i: (i, 0)),
            full((1, _H * _C)), full((1, _H * _C)),
            full((_H * _C, 32)), full((1, 32)), full((32, 2)), full((1, 2)),
        ],
        out_specs=pl.BlockSpec((_G, 2), lambda i: (0, 0)),
        out_shape=jax.ShapeDtypeStruct((_G, 2), jnp.float32),
        scratch_shapes=[
            pltpu.VMEM((_G, _H * _C), jnp.float32),
            pltpu.VMEM((_G, 8), jnp.float32),
        ],
    )(accumT, s2, batch2, s3, b3, w1, b1, w2, b2)


def kernel(x, edge_index, batch, ln_g, ln_b, bn1_g, bn1_b, bn1_m, bn1_v,
           cnn_W, cnn_b, bn2_g, bn2_b, bn2_m, bn2_v, pre_W, pre_b, gat_W,
           att_src, att_dst, gat_bias, bn3_g, bn3_b, bn3_m, bn3_v,
           mlp_W1, mlp_b1, mlp_W2, mlp_b2):
    f32 = jnp.float32
    s1 = bn1_g / jnp.sqrt(bn1_v + _EPS)
    b1 = bn1_b - bn1_m * s1
    c16_W = s1[:, None] * cnn_W
    c16_b = b1 @ cnn_W + cnn_b
    s2 = bn2_g / jnp.sqrt(bn2_v + _EPS)
    b2 = bn2_b - bn2_m * s2
    w_a = (c16_W * s2[None, :16]) @ pre_W[:16]
    w_b = (ln_g * s2[16:])[:, None] * pre_W[16:]
    b_ab = ((c16_b * s2[:16] + b2[:16]) @ pre_W[:16]
            + (ln_b * s2[16:] + b2[16:]) @ pre_W[16:] + pre_b)
    as_w = jnp.einsum("khc,hc->kh", gat_W.reshape(32, _H, _C), att_src)
    ad_w = jnp.einsum("khc,hc->kh", gat_W.reshape(32, _H, _C), att_dst)
    as_w = jnp.pad(as_w, ((0, 0), (0, 16 - _H)))
    ad_w = jnp.pad(ad_w, ((0, 0), (0, 16 - _H)))

    zT, as4, ad4, M = _pre_call(x, w_a, w_b, b_ab.reshape(1, 32), gat_W,
                                as_w, ad_w)

    t = M[0, :_H] + M[0, _H:]
    shift = jnp.where(t < 0, 0.2 * t, t)
    mrep = jnp.tile(shift, 4).astype(f32)

    ereal = edge_index.shape[1] + _N
    pad = _EPAD - ereal
    loops = jnp.arange(_N, dtype=jnp.int32)
    zpad = jnp.zeros((pad,), jnp.int32)
    src3 = jnp.concatenate([edge_index[0], loops, zpad]).reshape(_NCH, 8, 128)
    dst3 = jnp.concatenate([edge_index[1], loops, zpad]).reshape(_NCH, 8, 128)
    zflat = zT.reshape(2 * _H * _N, 16)

    accumT, s2p, _spill = _edge_call(ereal, src3, dst3, as4, ad4, zflat, mrep)
    s4 = s2p.reshape(_H, _NSP).T

    s3 = bn3_g / jnp.sqrt(bn3_v + _EPS)
    b3 = bn3_b + (gat_bias - bn3_m) * s3
    return _post_call(accumT, s4, batch.reshape(_N, 1),
                      s3.reshape(1, _H * _C), b3.reshape(1, _H * _C),
                      mlp_W1, mlp_b1.reshape(1, 32), mlp_W2,
                      mlp_b2.reshape(1, 2))


# speedup vs baseline: 32.7570x; 1.0420x over previous
import functools

import jax
import jax.numpy as jnp
from jax import lax
from jax.experimental import pallas as pl
from jax.experimental.pallas import tpu as pltpu
from jax.experimental.pallas import tpu_sc as plsc

_N = 50000
_H = 4
_C = 32
_G = 256
_EPS = 1e-5
_R = 2000
_GRID = _N // _R

_NSC = 2
_NTILE = 16
_CHUNK = 1024
_CPT = 52
_NCH = _NTILE * _CPT
_EPAD = _NCH * _CHUNK
_RPT = 3128
_NSP = _NTILE * _RPT


def _pre_body(x_ref, wa_ref, wb_ref, b_ref, gw_ref, as_w_ref, ad_w_ref,
              z_ref, as_ref, ad_ref, m_ref):
    i = pl.program_id(0)
    xb = x_ref[...]
    xc = xb[:, :512]
    xf = xb[:, 512:]
    mu = jnp.mean(xf, axis=-1, keepdims=True)
    var = jnp.mean((xf - mu) ** 2, axis=-1, keepdims=True)
    xfn = (xf - mu) / jnp.sqrt(var + _EPS)
    h = jnp.dot(xc, wa_ref[...], preferred_element_type=jnp.float32)
    h = h + jnp.dot(xfn, wb_ref[...], preferred_element_type=jnp.float32)
    h = jnp.maximum(h + b_ref[...], 0.0)
    z = jnp.dot(h, gw_ref[...], preferred_element_type=jnp.float32)
    for hd in range(_H):
        z_ref[hd] = z[:, hd * _C:(hd + 1) * _C]
    a_s = jnp.dot(h, as_w_ref[...], preferred_element_type=jnp.float32)
    a_d = jnp.dot(h, ad_w_ref[...], preferred_element_type=jnp.float32)
    as_ref[...] = a_s
    ad_ref[...] = a_d
    mx = jnp.concatenate([jnp.max(a_s, axis=0)[:_H], jnp.max(a_d, axis=0)[:_H]])
    mx = mx.reshape(1, 2 * _H)

    @pl.when(i == 0)
    def _():
        m_ref[...] = mx

    @pl.when(i != 0)
    def _():
        m_ref[...] = jnp.maximum(m_ref[...], mx)


def _pre_call(x, w_a, w_b, b_ab, gat_W, as_w, ad_w):
    full = lambda shape: pl.BlockSpec(shape, lambda i: (0,) * len(shape))
    return pl.pallas_call(
        _pre_body,
        grid=(_GRID,),
        in_specs=[
            pl.BlockSpec((_R, 576), lambda i: (i, 0)),
            full((512, 32)), full((64, 32)), full((1, 32)),
            full((32, _H * _C)), full((32, 16)), full((32, 16)),
        ],
        out_specs=[
            pl.BlockSpec((_H, _R, _C), lambda i: (0, i, 0)),
            pl.BlockSpec((_R, 16), lambda i: (i, 0)),
            pl.BlockSpec((_R, 16), lambda i: (i, 0)),
            pl.BlockSpec((1, 2 * _H), lambda i: (0, 0)),
        ],
        out_shape=[
            jax.ShapeDtypeStruct((_H, _N, _C), jnp.float32),
            jax.ShapeDtypeStruct((_N, 16), jnp.float32),
            jax.ShapeDtypeStruct((_N, 16), jnp.float32),
            jax.ShapeDtypeStruct((1, 2 * _H), jnp.float32),
        ],
    )(x, w_a, w_b, b_ab, gat_W, as_w, ad_w)


def _edge_body(ereal, src_h, dst_h, as_h, ad_h, zf_h, mrep_h,
               accum_o, s_o, exsp_o,
               src_v, dst_v, srcz_v, src2_v, dst2_v, srcz2_v,
               exh_v, exh1_v,
               zs_v, zrows_v, zrows2_v, zbuf_v, mrep_v, sem, semz, sems,
               accum_sp, s_sp):
    as_v = zrows2_v
    ad_v = zbuf_v
    cid = lax.axis_index("c")
    sid = lax.axis_index("s")
    iota = lax.iota(jnp.int32, 16)
    zeros16 = jnp.zeros((16,), jnp.float32)

    def _z0(e, _):
        zbuf_v[e, pl.ds(0, 16)] = zeros16
        return 0

    def _z1(j, _):
        zs_v[pl.ds(j * 16, 16)] = zeros16
        return 0
    lax.fori_loop(0, _CHUNK // 16, _z1, 0)

    pltpu.sync_copy(mrep_h, mrep_v)

    r0 = sid * _RPT

    def zero_accum():
        lax.fori_loop(0, _CHUNK, _z0, 0)
        for q in range(3):
            pltpu.sync_copy(zbuf_v, accum_sp.at[pl.ds(r0 + q * 1024, 1024), :])
        pltpu.sync_copy(zbuf_v.at[pl.ds(0, 56), :],
                        accum_sp.at[pl.ds(r0 + 3072, 56), :])

    def zero_s():
        for q in range(3):
            pltpu.sync_copy(zs_v, s_sp.at[pl.ds(r0 + q * 1024, 1024)])
        pltpu.sync_copy(zs_v.at[pl.ds(0, 56)], s_sp.at[pl.ds(r0 + 3072, 56)])

    zero_accum()
    zero_s()
    plsc.subcore_barrier()

    mrv = mrep_v[...]

    srcB = [src_v, src2_v]
    dstB = [dst_v, dst2_v]
    srczB = [srcz_v, srcz2_v]
    zrowsB = [zrows_v, zrows2_v]
    exhB = [exh_v, exh1_v]

    def _mk_off(srcb, srczb, zmul_off):
        def _off(j3, _):
            sv = srcb[j3 >> 3, pl.ds((j3 & 7) * 16, 16)]
            srczb[j3 >> 3, pl.ds((j3 & 7) * 16, 16)] = sv * 2 + zmul_off
            return 0
        return _off

    def _mk_scale(zrowsb, exhb):
        def _scale(i, _):
            for t in range(16):
                e = i * 16 + t
                exb = plsc.load_gather(exhb, [jnp.broadcast_to(e, (16,))])
                zrowsb[e, pl.ds(0, 16)] = zrowsb[e, pl.ds(0, 16)] * exb
            return 0
        return _scale

    for p in range(4):
        hp, ch = p >> 1, p & 1
        head = 2 * cid + hp
        zmul_off = 2 * head * _N + ch

        if p == 0:
            hm0 = iota == head
            hm1 = iota == (head + 1)

            def chunk_body(k, _, head=head, zmul_off=zmul_off,
                           hm0=hm0, hm1=hm1):
                cg = sid * _CPT + k
                pltpu.sync_copy(src_h.at[cg], src_v)
                pltpu.sync_copy(dst_h.at[cg], dst_v)

                lax.fori_loop(0, _CHUNK // 16,
                              _mk_off(src_v, srcz_v, zmul_off), 0)
                zdescs = [pltpu.async_copy(
                    zf_h.at[srcz_v.at[j]],
                    zrows_v.at[pl.ds(j * 128, 128), :], semz)
                    for j in range(8)]

                descs = []
                for j in range(8):
                    descs.append(pltpu.async_copy(
                        as_h.at[src_v.at[j]],
                        as_v.at[pl.ds(j * 128, 128), :], sem))
                    descs.append(pltpu.async_copy(
                        ad_h.at[dst_v.at[j]],
                        ad_v.at[pl.ds(j * 128, 128), :], sem))
                for d in descs:
                    d.wait()

                cbase = cg * _CHUNK

                def _cex(i, _):
                    for t in range(4):
                        e = i * 4 + t
                        a16 = as_v[e, pl.ds(0, 16)]
                        d16 = ad_v[e, pl.ds(0, 16)]
                        u = a16 + d16
                        u = jnp.where(u < 0.0, u * jnp.float32(0.2), u)
                        ex = jnp.exp(u - mrv)
                        eidx = jnp.broadcast_to(e, (16,))
                        ok = (cbase + eidx) < ereal
                        ex = jnp.where(ok, ex, 0.0)
                        plsc.store_scatter(exh_v, [eidx], ex, mask=hm0)
                        plsc.store_scatter(exh1_v, [eidx], ex, mask=hm1)
                    return 0
                lax.fori_loop(0, _CHUNK // 4, _cex, 0)
                pltpu.sync_copy(exh_v,
                                exsp_o.at[pl.ds((head * _NCH + cg) * _CHUNK,
                                                _CHUNK)])
                pltpu.sync_copy(exh1_v,
                                exsp_o.at[pl.ds(((head + 1) * _NCH + cg)
                                                * _CHUNK, _CHUNK)])

                for d in zdescs:
                    d.wait()
                lax.fori_loop(0, _CHUNK // 16, _mk_scale(zrows_v, exh_v), 0)

                descs = []
                for j in range(8):
                    descs.append(pltpu.async_copy(
                        zrows_v.at[pl.ds(j * 128, 128), :],
                        accum_sp.at[dst_v.at[j]], sems, add=True))
                    descs.append(pltpu.async_copy(
                        exh_v.at[pl.ds(j * 128, 128)],
                        s_sp.at[dst_v.at[j]], sems, add=True))
                for d in descs:
                    d.wait()
                return 0

            lax.fori_loop(0, _CPT, chunk_body, 0)
        else:
            def pair_body(i, _, p=p, head=head, zmul_off=zmul_off):
                lds = [None, None]
                for b in range(2):
                    cg = sid * _CPT + i * 2 + b
                    lds[b] = ([pltpu.async_copy(src_h.at[cg], srcB[b], sem),
                               pltpu.async_copy(dst_h.at[cg], dstB[b], sem),
                               pltpu.async_copy(
                                   exsp_o.at[pl.ds((head * _NCH + cg)
                                                   * _CHUNK, _CHUNK)],
                                   exhB[b], sem)], cg)
                zds = [None, None]
                for b in range(2):
                    for d in lds[b][0]:
                        d.wait()
                    lax.fori_loop(0, _CHUNK // 16,
                                  _mk_off(srcB[b], srczB[b], zmul_off), 0)
                    zds[b] = [pltpu.async_copy(
                        zf_h.at[srczB[b].at[j]],
                        zrowsB[b].at[pl.ds(j * 128, 128), :], semz)
                        for j in range(8)]
                sds = [None, None]
                for b in range(2):
                    for d in zds[b]:
                        d.wait()
                    lax.fori_loop(0, _CHUNK // 16,
                                  _mk_scale(zrowsB[b], exhB[b]), 0)
                    descs = []
                    for j in range(8):
                        descs.append(pltpu.async_copy(
                            zrowsB[b].at[pl.ds(j * 128, 128), :],
                            accum_sp.at[dstB[b].at[j]], sems, add=True))
                        if p == 2:
                            descs.append(pltpu.async_copy(
                                exhB[b].at[pl.ds(j * 128, 128)],
                                s_sp.at[dstB[b].at[j]], sems, add=True))
                    sds[b] = descs
                for b in range(2):
                    for d in sds[b]:
                        d.wait()
                return 0

            lax.fori_loop(0, _CPT // 2, pair_body, 0)

        plsc.subcore_barrier()

        for q in range(3):
            pltpu.sync_copy(accum_sp.at[pl.ds(r0 + q * 1024, 1024), :],
                            accum_o.at[head, ch, pl.ds(r0 + q * 1024, 1024), :])
        pltpu.sync_copy(accum_sp.at[pl.ds(r0 + 3072, 56), :],
                        accum_o.at[head, ch, pl.ds(r0 + 3072, 56), :])
        if p == 0 or p == 2:
            sbase = head * _NSP + r0
            for q in range(3):
                pltpu.sync_copy(s_sp.at[pl.ds(r0 + q * 1024, 1024)],
                                s_o.at[pl.ds(sbase + q * 1024, 1024)])
            pltpu.sync_copy(s_sp.at[pl.ds(r0 + 3072, 56)],
                            s_o.at[pl.ds(sbase + 3072, 56)])
        if p != 3:
            zero_accum()
            if p == 1:
                zero_s()
            plsc.subcore_barrier()


def _edge_call(ereal, src3, dst3, as4, ad4, zflat, mrep):
    mesh = plsc.VectorSubcoreMesh(core_axis_name="c", subcore_axis_name="s",
                                  num_cores=_NSC, num_subcores=_NTILE)
    kfn = pl.kernel(
        functools.partial(_edge_body, ereal),
        compiler_params=pltpu.CompilerParams(needs_layout_passes=False,
                                             use_tc_tiling_on_sc=False),
        out_type=(
            jax.ShapeDtypeStruct((_H, 2, _NSP, 16), jnp.float32),
            jax.ShapeDtypeStruct((_H * _NSP,), jnp.float32),
            jax.ShapeDtypeStruct((_H * _NCH * _CHUNK,), jnp.float32),
        ),
        mesh=mesh,
        scratch_types=[
            pltpu.VMEM((8, 128), jnp.int32),
            pltpu.VMEM((8, 128), jnp.int32),
            pltpu.VMEM((8, 128), jnp.int32),
            pltpu.VMEM((8, 128), jnp.int32),
            pltpu.VMEM((8, 128), jnp.int32),
            pltpu.VMEM((8, 128), jnp.int32),
            pltpu.VMEM((_CHUNK,), jnp.float32),
            pltpu.VMEM((_CHUNK,), jnp.float32),
            pltpu.VMEM((_CHUNK,), jnp.float32),
            pltpu.VMEM((_CHUNK, 16), jnp.float32),
            pltpu.VMEM((_CHUNK, 16), jnp.float32),
            pltpu.VMEM((_CHUNK, 16), jnp.float32),
            pltpu.VMEM((16,), jnp.float32),
            pltpu.SemaphoreType.DMA,
            pltpu.SemaphoreType.DMA,
            pltpu.SemaphoreType.DMA,
            pltpu.VMEM_SHARED((_NSP, 16), jnp.float32),
            pltpu.VMEM_SHARED((_NSP,), jnp.float32),
        ],
    )
    return kfn(src3, dst3, as4, ad4, zflat, mrep)


def _post_body(acc_ref, s_ref, b_ref, s3_ref, b3_ref,
               w1_ref, b1_ref, w2_ref, b2_ref,
               out_ref, pooled, cnt):
    i = pl.program_id(0)
    s = s_ref[...] + jnp.float32(1e-16)
    inv = 1.0 / s
    parts = []
    for hd in range(_H):
        vh = jnp.concatenate([acc_ref[hd, 0], acc_ref[hd, 1]], axis=1)
        parts.append(vh * inv[:, hd:hd + 1])
    v = jnp.concatenate(parts, axis=1)
    v = v * s3_ref[...] + b3_ref[...]
    v = jnp.where(v > 0.0, v, jnp.exp(v) - 1.0)

    g_iota = lax.broadcasted_iota(jnp.int32, (1, _G), 1)
    oh = (b_ref[...] == g_iota).astype(jnp.float32)
    dn = (((0,), (0,)), ((), ()))
    pool_inc = lax.dot_general(oh, v, dn, preferred_element_type=jnp.float32)
    cnt_inc = lax.dot_general(oh, jnp.ones((_R, 8), jnp.float32), dn,
                              preferred_element_type=jnp.float32)

    @pl.when(i == 0)
    def _():
        pooled[...] = pool_inc
        cnt[...] = cnt_inc

    @pl.when(i != 0)
    def _():
        pooled[...] = pooled[...] + pool_inc
        cnt[...] = cnt[...] + cnt_inc

    @pl.when(i == _GRID - 1)
    def _():
        mean = pooled[...] / jnp.maximum(cnt[...][:, :1], 1.0)
        t = jnp.dot(mean, w1_ref[...], preferred_element_type=jnp.float32)
        t = jnp.maximum(t + b1_ref[...], 0.0)
        out = jnp.dot(t, w2_ref[...], preferred_element_type=jnp.float32)
        out_ref[...] = out + b2_ref[...]


def _post_call(accumT, s2, batch2, s3, b3, w1, b1, w2, b2):
    full = lambda shape: pl.BlockSpec(shape, lambda i: (0,) * len(shape))
    return pl.pallas_call(
        _post_body,
        grid=(_GRID,),
        in_specs=[
            pl.BlockSpec((_H, 2, _R, 16), lambda i: (0, 0, i, 0)),
            pl.BlockSpec((_R, _H), lambda i: (i, 0)),
            pl.BlockSpec((_R, 1), lambda i: (i, 0)),
            full((1, _H * _C)), full((1, _H * _C)),
            full((_H * _C, 32)), full((1, 32)), full((32, 2)), full((1, 2)),
        ],
        out_specs=pl.BlockSpec((_G, 2), lambda i: (0, 0)),
        out_shape=jax.ShapeDtypeStruct((_G, 2), jnp.float32),
        scratch_shapes=[
            pltpu.VMEM((_G, _H * _C), jnp.float32),
            pltpu.VMEM((_G, 8), jnp.float32),
        ],
    )(accumT, s2, batch2, s3, b3, w1, b1, w2, b2)


def kernel(x, edge_index, batch, ln_g, ln_b, bn1_g, bn1_b, bn1_m, bn1_v,
           cnn_W, cnn_b, bn2_g, bn2_b, bn2_m, bn2_v, pre_W, pre_b, gat_W,
           att_src, att_dst, gat_bias, bn3_g, bn3_b, bn3_m, bn3_v,
           mlp_W1, mlp_b1, mlp_W2, mlp_b2):
    f32 = jnp.float32
    s1 = bn1_g / jnp.sqrt(bn1_v + _EPS)
    b1 = bn1_b - bn1_m * s1
    c16_W = s1[:, None] * cnn_W
    c16_b = b1 @ cnn_W + cnn_b
    s2 = bn2_g / jnp.sqrt(bn2_v + _EPS)
    b2 = bn2_b - bn2_m * s2
    w_a = (c16_W * s2[None, :16]) @ pre_W[:16]
    w_b = (ln_g * s2[16:])[:, None] * pre_W[16:]
    b_ab = ((c16_b * s2[:16] + b2[:16]) @ pre_W[:16]
            + (ln_b * s2[16:] + b2[16:]) @ pre_W[16:] + pre_b)
    as_w = jnp.einsum("khc,hc->kh", gat_W.reshape(32, _H, _C), att_src)
    ad_w = jnp.einsum("khc,hc->kh", gat_W.reshape(32, _H, _C), att_dst)
    as_w = jnp.pad(as_w, ((0, 0), (0, 16 - _H)))
    ad_w = jnp.pad(ad_w, ((0, 0), (0, 16 - _H)))

    zT, as4, ad4, M = _pre_call(x, w_a, w_b, b_ab.reshape(1, 32), gat_W,
                                as_w, ad_w)

    t = M[0, :_H] + M[0, _H:]
    shift = jnp.where(t < 0, 0.2 * t, t)
    mrep = jnp.tile(shift, 4).astype(f32)

    ereal = edge_index.shape[1] + _N
    pad = _EPAD - ereal
    loops = jnp.arange(_N, dtype=jnp.int32)
    zpad = jnp.zeros((pad,), jnp.int32)
    src3 = jnp.concatenate([edge_index[0], loops, zpad]).reshape(_NCH, 8, 128)
    dst3 = jnp.concatenate([edge_index[1], loops, zpad]).reshape(_NCH, 8, 128)
    zflat = zT.reshape(2 * _H * _N, 16)

    accumT, s2p, _spill = _edge_call(ereal, src3, dst3, as4, ad4, zflat, mrep)
    s4 = s2p.reshape(_H, _NSP).T

    s3 = bn3_g / jnp.sqrt(bn3_v + _EPS)
    b3 = bn3_b + (gat_bias - bn3_m) * s3
    return _post_call(accumT, s4, batch.reshape(_N, 1),
                      s3.reshape(1, _H * _C), b3.reshape(1, _H * _C),
                      mlp_W1, mlp_b1.reshape(1, 32), mlp_W2,
                      mlp_b2.reshape(1, 2))
